# baseline (speedup 1.0000x reference)
import functools

import jax
import jax.numpy as jnp
import numpy as np
from jax.experimental import pallas as pl
from jax.experimental.pallas import tpu as pltpu


def _row_lse_kernel(emb_ref, lse_ref):
    x = emb_ref[...]
    m = jnp.max(x, axis=1, keepdims=True)
    s = jnp.sum(jnp.exp(x - m), axis=1, keepdims=True)
    lse_ref[...] = (m + jnp.log(s))[:, :, None]


_HALVES = 8
_GROUP = 16 * _HALVES
_UNROLL_G = 1
_NACC = 4


def _gather_mm_kernel(tok_ref, lab_ref, p_ref, emb_ref, lse_ref,
                      out_ref, loss_ref, *, tn, sub, lane):
    base0 = pl.program_id(0) * tn
    flat = (lane * jax.lax.broadcasted_iota(jnp.int32, (sub, lane), 0)
            + jax.lax.broadcasted_iota(jnp.int32, (sub, lane), 1))
    pmat = p_ref[...]

    def body(gg, carry):
        accs_c, accs_l = carry
        accs_c, accs_l = list(accs_c), list(accs_l)
        o_mats, bases, slab_lists, tok_lists = [], [], [], []
        for u in range(_UNROLL_G):
            row_base = (gg * _UNROLL_G + u) * _GROUP
            bases.append(row_base)
            toks = [tok_ref[base0 + row_base + r] for r in range(_GROUP)]
            slabs = [emb_ref[t] for t in toks]
            tok_lists.append(toks)
            slab_lists.append(slabs)
            halves = [jnp.concatenate(slabs[16 * h:16 * (h + 1)], axis=0)
                      for h in range(_HALVES)]
            s_mat = jnp.concatenate(halves, axis=1)
            o_mats.append(jnp.dot(pmat, s_mat,
                                  preferred_element_type=jnp.float32))
        for u in range(_UNROLL_G):
            o_mat, row_base = o_mats[u], bases[u]
            for j in range(sub):
                for h in range(_HALVES):
                    for r8 in range(2):
                        orow = 16 * j + 8 * r8
                        dst = pl.multiple_of(row_base + 16 * h + 8 * r8, 8)
                        out_ref[pl.ds(dst, 8), lane * j:lane * (j + 1)] = (
                            o_mat[orow:orow + 8, lane * h:lane * (h + 1)])
        for u in range(_UNROLL_G):
            row_base = bases[u]
            for r in range(_GROUP):
                lbl = lab_ref[base0 + row_base + r]
                slab32 = slab_lists[u][r].astype(jnp.float32)
                k = r % _NACC
                accs_c[k] = accs_c[k] + jnp.where(flat == lbl, slab32, 0.0)
                accs_l[k] = accs_l[k] + lse_ref[tok_lists[u][r]]
        return tuple(accs_c), tuple(accs_l)

    accs_c = tuple(jnp.zeros((sub, lane), jnp.float32) for _ in range(_NACC))
    accs_l = tuple(jnp.zeros((1, 1), jnp.float32) for _ in range(_NACC))
    accs_c, accs_l = jax.lax.fori_loop(
        0, tn // (_GROUP * _UNROLL_G), body, (accs_c, accs_l))
    corr = sum(accs_c[1:], accs_c[0])
    lse_tot = sum(accs_l[1:], accs_l[0])
    part = (lse_tot - jnp.sum(corr, keepdims=True)[:1, :1]).reshape(1, 1, 1)

    @pl.when(pl.program_id(0) == 0)
    def _():
        loss_ref[...] = part

    @pl.when(pl.program_id(0) != 0)
    def _():
        loss_ref[...] = loss_ref[...] + part


def kernel(sequences, labels, emb_table):
    B, T = sequences.shape
    V = emb_table.shape[0]
    N = B * T
    LANE = 128
    SUB = V // LANE

    tok = sequences.reshape(N).astype(jnp.int32)
    lab = labels.reshape(N).astype(jnp.int32)
    emb = emb_table.astype(jnp.float32)
    emb_b3 = emb.astype(jnp.bfloat16).reshape(V, SUB, LANE)

    RB = min(512, V)
    lse3 = pl.pallas_call(
        _row_lse_kernel,
        out_shape=jax.ShapeDtypeStruct((V, 1, 1), jnp.float32),
        grid=(V // RB,),
        in_specs=[pl.BlockSpec((RB, V), lambda i: (i, 0))],
        out_specs=pl.BlockSpec((RB, 1, 1), lambda i: (i, 0, 0)),
        compiler_params=pltpu.CompilerParams(
            dimension_semantics=("parallel",)),
    )(emb)

    PM = 16 * SUB
    p_np = np.zeros((PM, PM), np.float32)
    o_np = np.arange(PM)
    p_np[o_np, SUB * (o_np % 16) + o_np // 16] = 1.0
    pmat = jnp.asarray(p_np, dtype=jnp.bfloat16)

    TN = 2048
    while N % TN:
        TN //= 2
    num_tiles = N // TN

    grid_spec = pltpu.PrefetchScalarGridSpec(
        num_scalar_prefetch=2,
        grid=(num_tiles,),
        in_specs=[
            pl.BlockSpec((PM, PM), lambda i, tok_s, lab_s: (0, 0)),
            pl.BlockSpec((V, SUB, LANE), lambda i, tok_s, lab_s: (0, 0, 0)),
            pl.BlockSpec((V, 1, 1), lambda i, tok_s, lab_s: (0, 0, 0)),
        ],
        out_specs=[
            pl.BlockSpec((TN, V), lambda i, tok_s, lab_s: (i, 0)),
            pl.BlockSpec((1, 1, 1), lambda i, tok_s, lab_s: (0, 0, 0)),
        ],
    )
    logits, loss_acc = pl.pallas_call(
        functools.partial(_gather_mm_kernel, tn=TN, sub=SUB, lane=LANE),
        grid_spec=grid_spec,
        out_shape=(
            jax.ShapeDtypeStruct((N, V), jnp.float32),
            jax.ShapeDtypeStruct((1, 1, 1), jnp.float32),
        ),
        compiler_params=pltpu.CompilerParams(
            dimension_semantics=("arbitrary",),
            vmem_limit_bytes=56 * 1024 * 1024),
    )(tok, lab, pmat, emb_b3, lse3)

    prediction_scores = logits.reshape(B, T, V)
    loss = loss_acc[0, 0, 0] / N
    return prediction_scores, loss


# speedup vs baseline: 1.0086x; 1.0086x over previous
import functools

import jax
import jax.numpy as jnp
import numpy as np
from jax.experimental import pallas as pl
from jax.experimental.pallas import tpu as pltpu


def _row_lse_kernel(emb_ref, lse_ref):
    x = emb_ref[...]
    m = jnp.max(x, axis=1, keepdims=True)
    s = jnp.sum(jnp.exp(x - m), axis=1, keepdims=True)
    lse_ref[...] = (m + jnp.log(s))[:, :, None]


_HALVES = 8
_GROUP = 16 * _HALVES
_UNROLL_G = 1
_NACC = 4


def _gather_mm_kernel(tok_ref, lab_ref, p_ref, emb_ref, lse_ref,
                      out_ref, loss_ref, *, tn, sub, lane):
    base0 = pl.program_id(0) * tn
    flat = (lane * jax.lax.broadcasted_iota(jnp.int32, (sub, lane), 0)
            + jax.lax.broadcasted_iota(jnp.int32, (sub, lane), 1))
    pmat = p_ref[...]

    def body(gg, carry):
        accs_c, accs_l = carry
        accs_c, accs_l = list(accs_c), list(accs_l)
        o_mats, bases, slab_lists, tok_lists = [], [], [], []
        for u in range(_UNROLL_G):
            row_base = (gg * _UNROLL_G + u) * _GROUP
            bases.append(row_base)
            toks = [tok_ref[base0 + row_base + r] for r in range(_GROUP)]
            slabs = [emb_ref[t] for t in toks]
            tok_lists.append(toks)
            slab_lists.append(slabs)
            halves = [jnp.concatenate(slabs[16 * h:16 * (h + 1)], axis=0)
                      for h in range(_HALVES)]
            s_mat = jnp.concatenate(halves, axis=1)
            o_mats.append(jnp.dot(pmat, s_mat,
                                  preferred_element_type=jnp.float32))
        for u in range(_UNROLL_G):
            o_mat, row_base = o_mats[u], bases[u]
            for j in range(sub):
                for h in range(_HALVES):
                    for r8 in range(2):
                        orow = 16 * j + 8 * r8
                        dst = pl.multiple_of(row_base + 16 * h + 8 * r8, 8)
                        out_ref[pl.ds(dst, 8), lane * j:lane * (j + 1)] = (
                            o_mat[orow:orow + 8, lane * h:lane * (h + 1)])
        for u in range(_UNROLL_G):
            row_base = bases[u]
            for r in range(_GROUP):
                lbl = lab_ref[base0 + row_base + r]
                slab32 = slab_lists[u][r].astype(jnp.float32)
                k = r % _NACC
                accs_c[k] = accs_c[k] + jnp.where(flat == lbl, slab32, 0.0)
                accs_l[k] = accs_l[k] + lse_ref[tok_lists[u][r]]
        return tuple(accs_c), tuple(accs_l)

    accs_c = tuple(jnp.zeros((sub, lane), jnp.float32) for _ in range(_NACC))
    accs_l = tuple(jnp.zeros((1, 1), jnp.float32) for _ in range(_NACC))
    accs_c, accs_l = jax.lax.fori_loop(
        0, tn // (_GROUP * _UNROLL_G), body, (accs_c, accs_l))
    corr = sum(accs_c[1:], accs_c[0])
    lse_tot = sum(accs_l[1:], accs_l[0])
    part = (lse_tot - jnp.sum(corr, keepdims=True)[:1, :1]).reshape(1, 1, 1)

    @pl.when(pl.program_id(0) == 0)
    def _():
        loss_ref[...] = part

    @pl.when(pl.program_id(0) != 0)
    def _():
        loss_ref[...] = loss_ref[...] + part


def kernel(sequences, labels, emb_table):
    B, T = sequences.shape
    V = emb_table.shape[0]
    N = B * T
    LANE = 128
    SUB = V // LANE

    tok = sequences.reshape(N).astype(jnp.int32)
    lab = labels.reshape(N).astype(jnp.int32)
    emb = emb_table.astype(jnp.float32)
    emb_b3 = emb.astype(jnp.bfloat16).reshape(V, SUB, LANE)

    RB = min(512, V)
    lse3 = pl.pallas_call(
        _row_lse_kernel,
        out_shape=jax.ShapeDtypeStruct((V, 1, 1), jnp.float32),
        grid=(V // RB,),
        in_specs=[pl.BlockSpec((RB, V), lambda i: (i, 0))],
        out_specs=pl.BlockSpec((RB, 1, 1), lambda i: (i, 0, 0)),
        compiler_params=pltpu.CompilerParams(
            dimension_semantics=("parallel",)),
    )(emb)

    PM = 16 * SUB
    p_np = np.zeros((PM, PM), np.float32)
    o_np = np.arange(PM)
    p_np[o_np, SUB * (o_np % 16) + o_np // 16] = 1.0
    pmat = jnp.asarray(p_np, dtype=jnp.bfloat16)

    TN = 1024
    while N % TN:
        TN //= 2
    num_tiles = N // TN

    grid_spec = pltpu.PrefetchScalarGridSpec(
        num_scalar_prefetch=2,
        grid=(num_tiles,),
        in_specs=[
            pl.BlockSpec((PM, PM), lambda i, tok_s, lab_s: (0, 0)),
            pl.BlockSpec((V, SUB, LANE), lambda i, tok_s, lab_s: (0, 0, 0)),
            pl.BlockSpec((V, 1, 1), lambda i, tok_s, lab_s: (0, 0, 0)),
        ],
        out_specs=[
            pl.BlockSpec((TN, V), lambda i, tok_s, lab_s: (i, 0)),
            pl.BlockSpec((1, 1, 1), lambda i, tok_s, lab_s: (0, 0, 0)),
        ],
    )
    logits, loss_acc = pl.pallas_call(
        functools.partial(_gather_mm_kernel, tn=TN, sub=SUB, lane=LANE),
        grid_spec=grid_spec,
        out_shape=(
            jax.ShapeDtypeStruct((N, V), jnp.float32),
            jax.ShapeDtypeStruct((1, 1, 1), jnp.float32),
        ),
        compiler_params=pltpu.CompilerParams(
            dimension_semantics=("arbitrary",),
            vmem_limit_bytes=56 * 1024 * 1024),
    )(tok, lab, pmat, emb_b3, lse3)

    prediction_scores = logits.reshape(B, T, V)
    loss = loss_acc[0, 0, 0] / N
    return prediction_scores, loss
